# baseline (speedup 1.0000x reference)
import functools

import jax
import jax.numpy as jnp
from jax.experimental import pallas as pl
from jax.experimental.pallas import tpu as pltpu


def _gat_body(x_ref, wfc_ref, alf_ref, arf_ref, bgat_ref, wproj_ref,
              bproj_ref, maskl_ref, maskr_ref, out_ref, *, H, D, NB):
    f32 = jnp.float32
    bf16 = jnp.bfloat16
    HD = H * D
    F = x_ref.shape[2]
    ones_row = jnp.ones((1, F), dtype=bf16)

    wfct = jnp.transpose(wfc_ref[...].astype(bf16))
    acomb = (alf_ref[...].astype(bf16) * maskl_ref[...]
             + arf_ref[...].astype(bf16) * maskr_ref[...])
    lcols = jax.lax.dot_general(wfct, acomb, (((1,), (0,)), ((), ())),
                                preferred_element_type=f32)
    wfcte = jnp.concatenate([wfct, lcols.astype(bf16)],
                            axis=1)
    wproj = wproj_ref[...].astype(bf16)
    bcol = jax.lax.dot_general(wproj, bgat_ref[...].astype(bf16),
                               (((1,), (0,)), ((), ())),
                               preferred_element_type=f32) + bproj_ref[...]

    for j in range(NB):
        nbT = jnp.transpose(x_ref[j].astype(bf16))
        feat_ext = jax.lax.dot_general(nbT, wfcte, (((1,), (0,)), ((), ())),
                                       preferred_element_type=f32)
        featb_ext = feat_ext.astype(bf16)
        featbT = jnp.transpose(featb_ext)
        el_maxs = jnp.max(feat_ext[:, HD:HD + H], axis=0,
                          keepdims=True)
        emax_col = jnp.transpose(el_maxs).astype(bf16)
        tt = emax_col + featbT[HD + H:HD + 2 * H, :]
        m_all = jnp.maximum(tt, 0.2 * tt)

        rst_rows = []
        for h in range(H):
            el_col = featb_ext[:, HD + h:HD + h + 1]
            er_row = featbT[HD + H + h:HD + H + h + 1, :]
            a = el_col + er_row
            e = jnp.maximum(a, 0.2 * a)
            p = jnp.exp(e - m_all[h:h + 1, :])
            lhs = jnp.concatenate(
                [featbT[h * D:(h + 1) * D, :], ones_row], axis=0)
            rq = jax.lax.dot_general(lhs, p, (((1,), (0,)), ((), ())),
                                     preferred_element_type=f32)
            r_row = 1.0 / rq[D:D + 1, :]
            rst_rows.append((rq[0:D, :] * r_row).astype(bf16))

        rstT = jnp.concatenate(rst_rows, axis=0)
        outT = jax.lax.dot_general(wproj, rstT, (((1,), (0,)), ((), ())),
                                   preferred_element_type=f32)
        out_ref[j] = outT + bcol


def kernel(x, W_fc, attn_l, attn_r, bias_gat, W_proj, b_proj):
    B, W, F = x.shape
    H, D = attn_l.shape
    HD = H * D

    f32 = jnp.float32
    bf16 = jnp.bfloat16
    heads = jax.lax.broadcasted_iota(jnp.int32, (HD, 2 * H), 0) // D
    cols = jax.lax.broadcasted_iota(jnp.int32, (HD, 2 * H), 1)
    maskl = (cols == heads).astype(bf16)
    maskr = (cols == heads + H).astype(bf16)

    NB = 16
    body = functools.partial(_gat_body, H=H, D=D, NB=NB)
    out = pl.pallas_call(
        body,
        grid=(B // NB,),
        in_specs=[
            pl.BlockSpec((NB, W, F), lambda b: (b, 0, 0)),
            pl.BlockSpec((HD, W), lambda b: (0, 0)),
            pl.BlockSpec((HD, 1), lambda b: (0, 0)),
            pl.BlockSpec((HD, 1), lambda b: (0, 0)),
            pl.BlockSpec((HD, 1), lambda b: (0, 0)),
            pl.BlockSpec((W, HD), lambda b: (0, 0)),
            pl.BlockSpec((W, 1), lambda b: (0, 0)),
            pl.BlockSpec((HD, 2 * H), lambda b: (0, 0)),
            pl.BlockSpec((HD, 2 * H), lambda b: (0, 0)),
        ],
        out_specs=pl.BlockSpec((NB, W, F), lambda b: (b, 0, 0)),
        out_shape=jax.ShapeDtypeStruct((B, W, F), f32),
        compiler_params=pltpu.CompilerParams(
            dimension_semantics=("parallel",)),
    )(x, W_fc, attn_l.reshape(HD, 1), attn_r.reshape(HD, 1),
      bias_gat.reshape(HD, 1), W_proj, b_proj.reshape(W, 1), maskl, maskr)
    return out


# speedup vs baseline: 1.3455x; 1.3455x over previous
import functools

import jax
import jax.numpy as jnp
from jax.experimental import pallas as pl
from jax.experimental.pallas import tpu as pltpu


def _gat_body(x_ref, wfc_ref, al_ref, ar_ref, bgat_ref, wproj_ref, bproj_ref,
              out_ref, *, H, D, NB):
    f32 = jnp.float32
    bf16 = jnp.bfloat16
    HD = H * D
    F = x_ref.shape[2]
    ones_row = jnp.ones((1, F), dtype=bf16)

    o_head = jax.lax.broadcasted_iota(jnp.int32, (HD, 2 * H), 0) // D
    o_col = jax.lax.broadcasted_iota(jnp.int32, (HD, 2 * H), 1)
    maskl = (o_col == o_head).astype(bf16)
    maskr = (o_col == o_head + H).astype(bf16)
    p1 = maskl[:, :H]
    d_row = jax.lax.broadcasted_iota(jnp.int32, (HD, D), 0) % D
    d_col = jax.lax.broadcasted_iota(jnp.int32, (HD, D), 1)
    dmask = (d_col == d_row).astype(bf16)
    ones_d = jnp.ones((D, 1), dtype=bf16)

    al_rep = jax.lax.dot_general(p1, al_ref[...].astype(bf16),
                                 (((1,), (0,)), ((), ())),
                                 preferred_element_type=f32)
    ar_rep = jax.lax.dot_general(p1, ar_ref[...].astype(bf16),
                                 (((1,), (0,)), ((), ())),
                                 preferred_element_type=f32)
    alf = jax.lax.dot_general(al_rep.astype(bf16) * dmask, ones_d,
                              (((1,), (0,)), ((), ())),
                              preferred_element_type=f32)
    arf = jax.lax.dot_general(ar_rep.astype(bf16) * dmask, ones_d,
                              (((1,), (0,)), ((), ())),
                              preferred_element_type=f32)
    acomb = (alf.astype(bf16) * maskl + arf.astype(bf16) * maskr)
    wfct = jnp.transpose(wfc_ref[...].astype(bf16))
    lcols = jax.lax.dot_general(wfct, acomb, (((1,), (0,)), ((), ())),
                                preferred_element_type=f32)
    wfcte = jnp.concatenate([wfct, lcols.astype(bf16)],
                            axis=1)
    wproj = wproj_ref[...].astype(bf16)
    bg_row = bgat_ref[...].reshape(1, HD).astype(bf16)
    brow = jax.lax.dot_general(bg_row, wproj, (((1,), (1,)), ((), ())),
                               preferred_element_type=f32)
    brow = brow + bproj_ref[...].reshape(1, -1)
    bcol = jnp.transpose(brow).astype(bf16)
    wproje = jnp.concatenate([wproj, bcol], axis=1)

    for j in range(NB):
        nbT = jnp.transpose(x_ref[j].astype(bf16))
        feat_ext = jax.lax.dot_general(nbT, wfcte, (((1,), (0,)), ((), ())),
                                       preferred_element_type=f32)
        featb_ext = feat_ext.astype(bf16)
        featbT = jnp.transpose(featb_ext)
        el_maxs = jnp.max(feat_ext[:, HD:HD + H], axis=0,
                          keepdims=True)
        emax_col = jnp.transpose(el_maxs).astype(bf16)
        tt = emax_col + featbT[HD + H:HD + 2 * H, :]
        m_all = jnp.maximum(tt, 0.2 * tt)

        rst_rows = []
        for h in range(H):
            el_col = featb_ext[:, HD + h:HD + h + 1]
            er_row = featbT[HD + H + h:HD + H + h + 1, :]
            a = el_col + er_row
            e = jnp.maximum(a, 0.2 * a)
            p = jnp.exp(e - m_all[h:h + 1, :])
            lhs = jnp.concatenate(
                [featbT[h * D:(h + 1) * D, :], ones_row], axis=0)
            rq = jax.lax.dot_general(lhs, p, (((1,), (0,)), ((), ())),
                                     preferred_element_type=f32)
            r_row = 1.0 / rq[D:D + 1, :]
            rst_rows.append((rq[0:D, :] * r_row).astype(bf16))

        rstT = jnp.concatenate(rst_rows + [ones_row], axis=0)
        outT = jax.lax.dot_general(wproje, rstT, (((1,), (0,)), ((), ())),
                                   preferred_element_type=f32)
        out_ref[j] = outT


def kernel(x, W_fc, attn_l, attn_r, bias_gat, W_proj, b_proj):
    B, W, F = x.shape
    H, D = attn_l.shape
    HD = H * D

    NB = 16
    body = functools.partial(_gat_body, H=H, D=D, NB=NB)
    out = pl.pallas_call(
        body,
        grid=(B // NB,),
        in_specs=[
            pl.BlockSpec((NB, W, F), lambda b: (b, 0, 0)),
            pl.BlockSpec((HD, W), lambda b: (0, 0)),
            pl.BlockSpec((H, D), lambda b: (0, 0)),
            pl.BlockSpec((H, D), lambda b: (0, 0)),
            pl.BlockSpec((HD,), lambda b: (0,)),
            pl.BlockSpec((W, HD), lambda b: (0, 0)),
            pl.BlockSpec((W,), lambda b: (0,)),
        ],
        out_specs=pl.BlockSpec((NB, W, F), lambda b: (b, 0, 0)),
        out_shape=jax.ShapeDtypeStruct((B, W, F), jnp.float32),
        compiler_params=pltpu.CompilerParams(
            dimension_semantics=("parallel",)),
    )(x, W_fc, attn_l, attn_r, bias_gat, W_proj, b_proj)
    return out


# speedup vs baseline: 1.3870x; 1.0308x over previous
import functools

import jax
import jax.numpy as jnp
from jax.experimental import pallas as pl
from jax.experimental.pallas import tpu as pltpu


def _gat_body(x_ref, wfc_ref, al_ref, ar_ref, bgat_ref, wproj_ref, bproj_ref,
              out_ref, *, H, D, NB):
    f32 = jnp.float32
    bf16 = jnp.bfloat16
    HD = H * D
    F = x_ref.shape[2]
    ones_row = jnp.ones((1, F), dtype=bf16)

    o_head = jax.lax.broadcasted_iota(jnp.int32, (HD, 2 * H), 0) // D
    o_col = jax.lax.broadcasted_iota(jnp.int32, (HD, 2 * H), 1)
    maskl = (o_col == o_head).astype(bf16)
    maskr = (o_col == o_head + H).astype(bf16)
    p1 = maskl[:, :H]
    d_row = jax.lax.broadcasted_iota(jnp.int32, (HD, D), 0) % D
    d_col = jax.lax.broadcasted_iota(jnp.int32, (HD, D), 1)
    dmask = (d_col == d_row).astype(bf16)
    ones_d = jnp.ones((D, 1), dtype=bf16)

    al_rep = jax.lax.dot_general(p1, al_ref[...].astype(bf16),
                                 (((1,), (0,)), ((), ())),
                                 preferred_element_type=f32)
    ar_rep = jax.lax.dot_general(p1, ar_ref[...].astype(bf16),
                                 (((1,), (0,)), ((), ())),
                                 preferred_element_type=f32)
    alf = jax.lax.dot_general(al_rep.astype(bf16) * dmask, ones_d,
                              (((1,), (0,)), ((), ())),
                              preferred_element_type=f32)
    arf = jax.lax.dot_general(ar_rep.astype(bf16) * dmask, ones_d,
                              (((1,), (0,)), ((), ())),
                              preferred_element_type=f32)
    acomb = (alf.astype(bf16) * maskl + arf.astype(bf16) * maskr)
    wfct = jnp.transpose(wfc_ref[...].astype(bf16))
    lcols = jax.lax.dot_general(wfct, acomb, (((1,), (0,)), ((), ())),
                                preferred_element_type=f32)
    wfcte = jnp.concatenate([wfct, lcols.astype(bf16)],
                            axis=1)
    wproj = wproj_ref[...].astype(bf16)
    bg_row = bgat_ref[...].reshape(1, HD).astype(bf16)
    brow = jax.lax.dot_general(bg_row, wproj, (((1,), (1,)), ((), ())),
                               preferred_element_type=f32)
    brow = brow + bproj_ref[...].reshape(1, -1)
    bcol = jnp.transpose(brow).astype(bf16)
    wproje = jnp.concatenate([wproj, bcol], axis=1)

    for j in range(NB):
        nbT = jnp.transpose(x_ref[j].astype(bf16))
        feat_ext = jax.lax.dot_general(nbT, wfcte, (((1,), (0,)), ((), ())),
                                       preferred_element_type=f32)
        featb_ext = feat_ext.astype(bf16)
        featbT = jnp.transpose(featb_ext)
        el_maxs = jnp.max(feat_ext[:, HD:HD + H], axis=0,
                          keepdims=True)
        emax_col = jnp.transpose(el_maxs).astype(bf16)
        tt = emax_col + featbT[HD + H:HD + 2 * H, :]
        m_all = jnp.maximum(tt, 0.2 * tt)

        rst_rows = []
        for h in range(H):
            el_col = featb_ext[:, HD + h:HD + h + 1]
            er_row = featbT[HD + H + h:HD + H + h + 1, :]
            a = el_col + er_row
            e = jnp.maximum(a, 0.2 * a)
            p = jnp.exp(e - m_all[h:h + 1, :])
            lhs = jnp.concatenate(
                [featbT[h * D:(h + 1) * D, :], ones_row], axis=0)
            rq = jax.lax.dot_general(lhs, p, (((1,), (0,)), ((), ())),
                                     preferred_element_type=f32)
            r_row = 1.0 / rq[D:D + 1, :]
            rst_rows.append((rq[0:D, :] * r_row).astype(bf16))

        rstT = jnp.concatenate(rst_rows + [ones_row], axis=0)
        outT = jax.lax.dot_general(wproje, rstT, (((1,), (0,)), ((), ())),
                                   preferred_element_type=f32)
        out_ref[j] = outT


def kernel(x, W_fc, attn_l, attn_r, bias_gat, W_proj, b_proj):
    B, W, F = x.shape
    H, D = attn_l.shape
    HD = H * D

    NB = 32
    body = functools.partial(_gat_body, H=H, D=D, NB=NB)
    out = pl.pallas_call(
        body,
        grid=(B // NB,),
        in_specs=[
            pl.BlockSpec((NB, W, F), lambda b: (b, 0, 0)),
            pl.BlockSpec((HD, W), lambda b: (0, 0)),
            pl.BlockSpec((H, D), lambda b: (0, 0)),
            pl.BlockSpec((H, D), lambda b: (0, 0)),
            pl.BlockSpec((HD,), lambda b: (0,)),
            pl.BlockSpec((W, HD), lambda b: (0, 0)),
            pl.BlockSpec((W,), lambda b: (0,)),
        ],
        out_specs=pl.BlockSpec((NB, W, F), lambda b: (b, 0, 0)),
        out_shape=jax.ShapeDtypeStruct((B, W, F), jnp.float32),
        compiler_params=pltpu.CompilerParams(
            dimension_semantics=("parallel",)),
    )(x, W_fc, attn_l, attn_r, bias_gat, W_proj, b_proj)
    return out


# speedup vs baseline: 1.4334x; 1.0334x over previous
import functools

import jax
import jax.numpy as jnp
from jax.experimental import pallas as pl
from jax.experimental.pallas import tpu as pltpu


def _gat_body(x_ref, wfc_ref, al_ref, ar_ref, bgat_ref, wproj_ref, bproj_ref,
              out_ref, *, H, D, NB):
    f32 = jnp.float32
    bf16 = jnp.bfloat16
    HD = H * D
    F = x_ref.shape[2]
    ones_row = jnp.ones((1, F), dtype=bf16)

    o_head = jax.lax.broadcasted_iota(jnp.int32, (HD, 2 * H), 0) // D
    o_col = jax.lax.broadcasted_iota(jnp.int32, (HD, 2 * H), 1)
    maskl = (o_col == o_head).astype(bf16)
    maskr = (o_col == o_head + H).astype(bf16)
    p1 = maskl[:, :H]
    d_row = jax.lax.broadcasted_iota(jnp.int32, (HD, D), 0) % D
    d_col = jax.lax.broadcasted_iota(jnp.int32, (HD, D), 1)
    dmask = (d_col == d_row).astype(bf16)
    ones_d = jnp.ones((D, 1), dtype=bf16)

    al_rep = jax.lax.dot_general(p1, al_ref[...].astype(bf16),
                                 (((1,), (0,)), ((), ())),
                                 preferred_element_type=f32)
    ar_rep = jax.lax.dot_general(p1, ar_ref[...].astype(bf16),
                                 (((1,), (0,)), ((), ())),
                                 preferred_element_type=f32)
    alf = jax.lax.dot_general(al_rep.astype(bf16) * dmask, ones_d,
                              (((1,), (0,)), ((), ())),
                              preferred_element_type=f32)
    arf = jax.lax.dot_general(ar_rep.astype(bf16) * dmask, ones_d,
                              (((1,), (0,)), ((), ())),
                              preferred_element_type=f32)
    acomb = (alf.astype(bf16) * maskl + arf.astype(bf16) * maskr)
    wfcb = wfc_ref[...].astype(bf16)
    acombT = jnp.transpose(acomb)
    lrows = jax.lax.dot_general(acombT, wfcb, (((1,), (0,)), ((), ())),
                                preferred_element_type=f32)
    wfcteT = jnp.concatenate([wfcb, lrows.astype(bf16)],
                             axis=0)
    wproj = wproj_ref[...].astype(bf16)
    bg_row = bgat_ref[...].reshape(1, HD).astype(bf16)
    brow = jax.lax.dot_general(bg_row, wproj, (((1,), (1,)), ((), ())),
                               preferred_element_type=f32)
    brow = brow + bproj_ref[...].reshape(1, -1)
    bcol = jnp.transpose(brow).astype(bf16)
    wproje = jnp.concatenate([wproj, bcol], axis=1)

    for j in range(NB):
        xb = x_ref[j].astype(bf16)
        featT_ext = jax.lax.dot_general(wfcteT, xb, (((1,), (0,)), ((), ())),
                                        preferred_element_type=f32)
        featbT = featT_ext.astype(bf16)
        el_colsT = jnp.transpose(featbT[HD:HD + H, :])
        el_maxs = jnp.max(featT_ext[HD:HD + H, :], axis=1,
                          keepdims=True)
        emax_col = el_maxs.astype(bf16)
        tt = emax_col + featbT[HD + H:HD + 2 * H, :]
        m_all = jnp.maximum(tt, 0.2 * tt)

        rst_rows = []
        for h in range(H):
            el_col = el_colsT[:, h:h + 1]
            er_row = featbT[HD + H + h:HD + H + h + 1, :]
            a = el_col + er_row
            e = jnp.maximum(a, 0.2 * a)
            p = jnp.exp(e - m_all[h:h + 1, :])
            lhs = jnp.concatenate(
                [featbT[h * D:(h + 1) * D, :], ones_row], axis=0)
            rq = jax.lax.dot_general(lhs, p, (((1,), (0,)), ((), ())),
                                     preferred_element_type=f32)
            r_row = 1.0 / rq[D:D + 1, :]
            rst_rows.append((rq[0:D, :] * r_row).astype(bf16))

        rstT = jnp.concatenate(rst_rows + [ones_row], axis=0)
        outT = jax.lax.dot_general(wproje, rstT, (((1,), (0,)), ((), ())),
                                   preferred_element_type=f32)
        out_ref[j] = outT


def kernel(x, W_fc, attn_l, attn_r, bias_gat, W_proj, b_proj):
    B, W, F = x.shape
    H, D = attn_l.shape
    HD = H * D

    NB = 32
    body = functools.partial(_gat_body, H=H, D=D, NB=NB)
    out = pl.pallas_call(
        body,
        grid=(B // NB,),
        in_specs=[
            pl.BlockSpec((NB, W, F), lambda b: (b, 0, 0)),
            pl.BlockSpec((HD, W), lambda b: (0, 0)),
            pl.BlockSpec((H, D), lambda b: (0, 0)),
            pl.BlockSpec((H, D), lambda b: (0, 0)),
            pl.BlockSpec((HD,), lambda b: (0,)),
            pl.BlockSpec((W, HD), lambda b: (0, 0)),
            pl.BlockSpec((W,), lambda b: (0,)),
        ],
        out_specs=pl.BlockSpec((NB, W, F), lambda b: (b, 0, 0)),
        out_shape=jax.ShapeDtypeStruct((B, W, F), jnp.float32),
        compiler_params=pltpu.CompilerParams(
            dimension_semantics=("parallel",)),
    )(x, W_fc, attn_l, attn_r, bias_gat, W_proj, b_proj)
    return out


# speedup vs baseline: 1.9311x; 1.3472x over previous
import functools

import jax
import jax.numpy as jnp
from jax.experimental import pallas as pl
from jax.experimental.pallas import tpu as pltpu


def _gat_body(x_ref, wfc_ref, al_ref, ar_ref, bgat_ref, wproj_ref, bproj_ref,
              out_ref, *, H, D, NB):
    f32 = jnp.float32
    bf16 = jnp.bfloat16
    HD = H * D
    F = x_ref.shape[2]
    ones_row = jnp.ones((1, F), dtype=bf16)

    o_head = jax.lax.broadcasted_iota(jnp.int32, (HD, 2 * H), 0) // D
    o_col = jax.lax.broadcasted_iota(jnp.int32, (HD, 2 * H), 1)
    maskl = (o_col == o_head).astype(bf16)
    maskr = (o_col == o_head + H).astype(bf16)
    p1 = maskl[:, :H]
    d_row = jax.lax.broadcasted_iota(jnp.int32, (HD, D), 0) % D
    d_col = jax.lax.broadcasted_iota(jnp.int32, (HD, D), 1)
    dmask = (d_col == d_row).astype(bf16)
    ones_d = jnp.ones((D, 1), dtype=bf16)

    al_rep = jax.lax.dot_general(p1, al_ref[...].astype(bf16),
                                 (((1,), (0,)), ((), ())),
                                 preferred_element_type=f32)
    ar_rep = jax.lax.dot_general(p1, ar_ref[...].astype(bf16),
                                 (((1,), (0,)), ((), ())),
                                 preferred_element_type=f32)
    alf = jax.lax.dot_general(al_rep.astype(bf16) * dmask, ones_d,
                              (((1,), (0,)), ((), ())),
                              preferred_element_type=f32)
    arf = jax.lax.dot_general(ar_rep.astype(bf16) * dmask, ones_d,
                              (((1,), (0,)), ((), ())),
                              preferred_element_type=f32)
    acomb = (alf.astype(bf16) * maskl + arf.astype(bf16) * maskr)
    wfcb = wfc_ref[...].astype(bf16)
    acombT = jnp.transpose(acomb)
    lrows = jax.lax.dot_general(acombT, wfcb, (((1,), (0,)), ((), ())),
                                preferred_element_type=f32)
    wfcteT = jnp.concatenate([wfcb, lrows.astype(bf16)],
                             axis=0)
    wproj = wproj_ref[...].astype(bf16)
    bg_row = bgat_ref[...].reshape(1, HD).astype(bf16)
    brow = jax.lax.dot_general(bg_row, wproj, (((1,), (1,)), ((), ())),
                               preferred_element_type=f32)
    brow = brow + bproj_ref[...].reshape(1, -1)
    bcol = jnp.transpose(brow).astype(bf16)
    wproje = jnp.concatenate([wproj, bcol], axis=1)

    def _featT(j):
        xb = x_ref[j].astype(bf16)
        return jax.lax.dot_general(wfcteT, xb, (((1,), (0,)), ((), ())),
                                   preferred_element_type=f32)

    featT_next = _featT(0)
    for j in range(NB):
        featT_ext = featT_next
        if j + 1 < NB:
            featT_next = _featT(j + 1)
        featbT = featT_ext.astype(bf16)
        el_colsT = jnp.transpose(featbT[HD:HD + H, :])
        el_maxs = jnp.max(featT_ext[HD:HD + H, :], axis=1,
                          keepdims=True)
        emax_col = el_maxs.astype(bf16)
        tt = emax_col + featbT[HD + H:HD + 2 * H, :]
        m_all = jnp.maximum(tt, 0.2 * tt)

        rst_rows = []
        for h in range(H):
            el_col = el_colsT[:, h:h + 1]
            er_row = featbT[HD + H + h:HD + H + h + 1, :]
            a = el_col + er_row
            e = jnp.maximum(a, 0.2 * a)
            p = jnp.exp(e - m_all[h:h + 1, :])
            lhs = jnp.concatenate(
                [featbT[h * D:(h + 1) * D, :], ones_row], axis=0)
            rq = jax.lax.dot_general(lhs, p, (((1,), (0,)), ((), ())),
                                     preferred_element_type=f32)
            r_row = 1.0 / rq[D:D + 1, :]
            rst_rows.append((rq[0:D, :] * r_row).astype(bf16))

        rstT = jnp.concatenate(rst_rows + [ones_row], axis=0)
        outT = jax.lax.dot_general(wproje, rstT, (((1,), (0,)), ((), ())),
                                   preferred_element_type=f32)
        out_ref[j] = outT


def kernel(x, W_fc, attn_l, attn_r, bias_gat, W_proj, b_proj):
    B, W, F = x.shape
    H, D = attn_l.shape
    HD = H * D

    NB = 32
    body = functools.partial(_gat_body, H=H, D=D, NB=NB)
    out = pl.pallas_call(
        body,
        grid=(B // NB,),
        in_specs=[
            pl.BlockSpec((NB, W, F), lambda b: (b, 0, 0)),
            pl.BlockSpec((HD, W), lambda b: (0, 0)),
            pl.BlockSpec((H, D), lambda b: (0, 0)),
            pl.BlockSpec((H, D), lambda b: (0, 0)),
            pl.BlockSpec((HD,), lambda b: (0,)),
            pl.BlockSpec((W, HD), lambda b: (0, 0)),
            pl.BlockSpec((W,), lambda b: (0,)),
        ],
        out_specs=pl.BlockSpec((NB, W, F), lambda b: (b, 0, 0)),
        out_shape=jax.ShapeDtypeStruct((B, W, F), jnp.float32),
        compiler_params=pltpu.CompilerParams(
            dimension_semantics=("parallel",)),
    )(x, W_fc, attn_l, attn_r, bias_gat, W_proj, b_proj)
    return out


# speedup vs baseline: 1.9659x; 1.0180x over previous
import functools

import jax
import jax.numpy as jnp
from jax.experimental import pallas as pl
from jax.experimental.pallas import tpu as pltpu


def _gat_body(x_ref, wfc_ref, al_ref, ar_ref, bgat_ref, wproj_ref, bproj_ref,
              out_ref, *, H, D, NB):
    f32 = jnp.float32
    bf16 = jnp.bfloat16
    HD = H * D
    F = x_ref.shape[2]
    ones_row = jnp.ones((1, F), dtype=bf16)

    o_head = jax.lax.broadcasted_iota(jnp.int32, (HD, 2 * H), 0) // D
    o_col = jax.lax.broadcasted_iota(jnp.int32, (HD, 2 * H), 1)
    maskl = (o_col == o_head).astype(bf16)
    maskr = (o_col == o_head + H).astype(bf16)
    p1 = maskl[:, :H]
    d_row = jax.lax.broadcasted_iota(jnp.int32, (HD, D), 0) % D
    d_col = jax.lax.broadcasted_iota(jnp.int32, (HD, D), 1)
    dmask = (d_col == d_row).astype(bf16)
    ones_d = jnp.ones((D, 1), dtype=bf16)

    al_rep = jax.lax.dot_general(p1, al_ref[...].astype(bf16),
                                 (((1,), (0,)), ((), ())),
                                 preferred_element_type=f32)
    ar_rep = jax.lax.dot_general(p1, ar_ref[...].astype(bf16),
                                 (((1,), (0,)), ((), ())),
                                 preferred_element_type=f32)
    alf = jax.lax.dot_general(al_rep.astype(bf16) * dmask, ones_d,
                              (((1,), (0,)), ((), ())),
                              preferred_element_type=f32)
    arf = jax.lax.dot_general(ar_rep.astype(bf16) * dmask, ones_d,
                              (((1,), (0,)), ((), ())),
                              preferred_element_type=f32)
    acomb = (alf.astype(bf16) * maskl + arf.astype(bf16) * maskr)
    wfcb = wfc_ref[...].astype(bf16)
    acombT = jnp.transpose(acomb)
    lrows = jax.lax.dot_general(acombT, wfcb, (((1,), (0,)), ((), ())),
                                preferred_element_type=f32)
    wfcteT = jnp.concatenate([wfcb, lrows.astype(bf16)],
                             axis=0)
    wproj = wproj_ref[...].astype(bf16)
    bg_row = bgat_ref[...].reshape(1, HD).astype(bf16)
    brow = jax.lax.dot_general(bg_row, wproj, (((1,), (1,)), ((), ())),
                               preferred_element_type=f32)
    brow = brow + bproj_ref[...].reshape(1, -1)
    bcol = jnp.transpose(brow).astype(bf16)
    wproje = jnp.concatenate([wproj, bcol], axis=1)

    def _featT(j):
        xb = x_ref[j].astype(bf16)
        return jax.lax.dot_general(wfcteT, xb, (((1,), (0,)), ((), ())),
                                   preferred_element_type=f32)

    pipe = [_featT(0), _featT(1)]
    for j in range(NB):
        featT_ext = pipe[0]
        pipe = pipe[1:]
        if j + 2 < NB:
            pipe.append(_featT(j + 2))
        featbT = featT_ext.astype(bf16)
        el_colsT = jnp.transpose(featbT[HD:HD + H, :])
        el_maxs = jnp.max(featT_ext[HD:HD + H, :], axis=1,
                          keepdims=True)
        emax_col = el_maxs.astype(bf16)
        tt = emax_col + featbT[HD + H:HD + 2 * H, :]
        m_all = jnp.maximum(tt, 0.2 * tt)

        rst_rows = []
        for h in range(H):
            el_col = el_colsT[:, h:h + 1]
            er_row = featbT[HD + H + h:HD + H + h + 1, :]
            a = el_col + er_row
            e = jnp.maximum(a, 0.2 * a)
            p = jnp.exp(e - m_all[h:h + 1, :])
            lhs = jnp.concatenate(
                [featbT[h * D:(h + 1) * D, :], ones_row], axis=0)
            rq = jax.lax.dot_general(lhs, p, (((1,), (0,)), ((), ())),
                                     preferred_element_type=f32)
            r_row = 1.0 / rq[D:D + 1, :]
            rst_rows.append((rq[0:D, :] * r_row).astype(bf16))

        rstT = jnp.concatenate(rst_rows + [ones_row], axis=0)
        outT = jax.lax.dot_general(wproje, rstT, (((1,), (0,)), ((), ())),
                                   preferred_element_type=f32)
        out_ref[j] = outT


def kernel(x, W_fc, attn_l, attn_r, bias_gat, W_proj, b_proj):
    B, W, F = x.shape
    H, D = attn_l.shape
    HD = H * D

    NB = 32
    body = functools.partial(_gat_body, H=H, D=D, NB=NB)
    out = pl.pallas_call(
        body,
        grid=(B // NB,),
        in_specs=[
            pl.BlockSpec((NB, W, F), lambda b: (b, 0, 0)),
            pl.BlockSpec((HD, W), lambda b: (0, 0)),
            pl.BlockSpec((H, D), lambda b: (0, 0)),
            pl.BlockSpec((H, D), lambda b: (0, 0)),
            pl.BlockSpec((HD,), lambda b: (0,)),
            pl.BlockSpec((W, HD), lambda b: (0, 0)),
            pl.BlockSpec((W,), lambda b: (0,)),
        ],
        out_specs=pl.BlockSpec((NB, W, F), lambda b: (b, 0, 0)),
        out_shape=jax.ShapeDtypeStruct((B, W, F), jnp.float32),
        compiler_params=pltpu.CompilerParams(
            dimension_semantics=("parallel",)),
    )(x, W_fc, attn_l, attn_r, bias_gat, W_proj, b_proj)
    return out
